# baseline (speedup 1.0000x reference)
import jax
import jax.numpy as jnp
from jax.experimental import pallas as pl

N = 10000
E = 320000
L = 16
EPS = 1e-15


def _bias_relu_kernel(x_ref, b_ref, o_ref):
    o_ref[...] = jnp.maximum(x_ref[...] + b_ref[...], 0.0)


def _bias_relu(x, b):
    return pl.pallas_call(
        _bias_relu_kernel,
        out_shape=jax.ShapeDtypeStruct(x.shape, x.dtype),
    )(x, jnp.broadcast_to(b, x.shape))


def _hconv(x, ei, W):
    xw = x @ W.T
    node = ei[0]
    he = ei[1]
    ones = jnp.ones((ei.shape[1],), dtype=x.dtype)
    deg = jax.ops.segment_sum(ones, node, num_segments=N)
    Dinv = jnp.where(deg > 0, 1.0 / deg, 0.0)
    bdeg = jax.ops.segment_sum(ones, he, num_segments=N)
    Binv = jnp.where(bdeg > 0, 1.0 / bdeg, 0.0)
    out_e = jax.ops.segment_sum(Binv[he][:, None] * xw[node], he, num_segments=N)
    out_n = jax.ops.segment_sum(Dinv[node][:, None] * out_e[he], node, num_segments=N)
    return out_n


def _hgnn(x, ei, W1, b1, W2, b2):
    x = _bias_relu(_hconv(x, ei, W1), b1)
    x = _bias_relu(_hconv(x, ei, W2), b2)
    return x


def _gen_hye(emb, hlen, hnode):
    table = jnp.concatenate([jnp.zeros((1, emb.shape[1]), dtype=emb.dtype), emb], axis=0)
    seq = table[hnode]
    return jnp.sum(seq, axis=1) / (hlen + EPS)


def _fusion(z, W1, b1, W2):
    w = (jnp.tanh(z @ W1.T + b1) @ W2.T).mean(0)
    beta = jax.nn.softmax(w, axis=0)
    return (beta[None, :, :] * z).sum(1)


def kernel(x0, x1, x2, x3, edge0, edge1, edge2, edge3, hlen0, hlen1, hlen2, hlen3, hnode0, hnode1, hnode2, hnode3, conv1_W, conv1_b, conv2_W, conv2_b, attnc_W1, attnc_b1, attnc_W2, attnm_W1, attnm_b1, attnm_W2):
    x1_cm = _hgnn(x0, edge0, conv1_W, conv1_b, conv2_W, conv2_b)
    x1_mc = _hgnn(x1, edge1, conv1_W, conv1_b, conv2_W, conv2_b)
    x1_cc = _hgnn(x2, edge2, conv1_W, conv1_b, conv2_W, conv2_b)
    x1_mm = _hgnn(x3, edge3, conv1_W, conv1_b, conv2_W, conv2_b)
    x2_cm = _gen_hye(x1_cm, hlen0, hnode0)
    x2_mc = _gen_hye(x1_mc, hlen1, hnode1)
    x2_cc = _gen_hye(x1_cc, hlen2, hnode2)
    x2_mm = _gen_hye(x1_mm, hlen3, hnode3)
    x_c = jnp.stack((x1_mc, x2_cm, x1_cc, x2_cc), axis=1)
    x_m = jnp.stack((x1_cm, x2_mc, x1_mm, x2_mm), axis=1)
    h_c = _fusion(x_c, attnc_W1, attnc_b1, attnc_W2)
    h_m = _fusion(x_m, attnm_W1, attnm_b1, attnm_W2)
    return (h_c, h_m)


# speedup vs baseline: 2.1562x; 2.1562x over previous
import functools

import jax
import jax.numpy as jnp
from jax import lax
from jax.experimental import pallas as pl
from jax.experimental.pallas import tpu as pltpu
from jax.experimental.pallas import tpu_sc as plsc

N = 10000
E = 320000
L = 16
EPS = 1e-15

NC = 2
NS = 16
CHUNK = 1024
KSUB = CHUNK // 128
ACC_ROWS = NS * 632
E_PAD = NS * 20 * CHUNK
HYE_PAD = NS * 10 * CHUNK


def _make_epass(table_rows, dh, nchunks):
    mesh = plsc.VectorSubcoreMesh(
        core_axis_name="c", subcore_axis_name="s", num_cores=NC, num_subcores=NS)

    @functools.partial(
        pl.kernel,
        out_type=jax.ShapeDtypeStruct((NC, ACC_ROWS, dh), jnp.float32),
        mesh=mesh,
        scratch_types=[
            pltpu.VMEM((KSUB, 128), jnp.int32),
            pltpu.VMEM((KSUB, 128), jnp.int32),
            pltpu.VMEM((CHUNK, dh), jnp.float32),
            pltpu.VMEM_SHARED((ACC_ROWS, dh), jnp.float32),
            pltpu.SemaphoreType.DMA,
        ],
        compiler_params=pltpu.CompilerParams(use_tc_tiling_on_sc=False),
    )
    def epass(table, gidx, sidx, zeros, out, gi_v, si_v, rows_v, acc, sem):
        c = lax.axis_index("c")
        s = lax.axis_index("s")
        zrows = ACC_ROWS // NS
        pltpu.sync_copy(zeros.at[pl.ds(s * zrows, zrows)],
                        acc.at[pl.ds(s * zrows, zrows)])
        plsc.subcore_barrier()

        def chunk_body(i, carry):
            cid = s * nchunks + i
            pltpu.sync_copy(gidx.at[c, cid], gi_v)
            pltpu.sync_copy(sidx.at[c, cid], si_v)
            descs = [
                pltpu.async_copy(table.at[gi_v.at[j]],
                                 rows_v.at[pl.ds(j * 128, 128)], sem)
                for j in range(KSUB)
            ]
            for d in descs:
                d.wait()
            descs = [
                pltpu.async_copy(rows_v.at[pl.ds(j * 128, 128)],
                                 acc.at[si_v.at[j]], sem, add=True)
                for j in range(KSUB)
            ]
            for d in descs:
                d.wait()
            return carry

        lax.fori_loop(0, nchunks, chunk_body, 0)
        plsc.subcore_barrier()
        pltpu.sync_copy(acc.at[pl.ds(s * zrows, zrows)],
                        out.at[c, pl.ds(s * zrows, zrows)])

    return epass


_epass64 = _make_epass(2 * N, 64, E_PAD // NS // CHUNK)
_epass16 = _make_epass(2 * N, 16, E_PAD // NS // CHUNK)
_epassdeg = _make_epass(8, 16, E_PAD // NS // CHUNK)
_epasshye = _make_epass(2 * (N + 1), 16, HYE_PAD // NS // CHUNK)


def _pad1(a, n, val):
    return jnp.concatenate([a, jnp.full((n - a.shape[0],), val, a.dtype)])


def _prep(idx2):
    return idx2.reshape(NC, -1, KSUB, 128)


def _graph_pipeline(x, ei, hlen, hnode, W1, b1, W2, b2, zeros64, zeros16,
                    ones_table, hye_sidx):
    node = ei[0]
    he = ei[1]
    node_g = _pad1(node, E_PAD, 0)
    node_s = _pad1(node, E_PAD, N)
    he_g = _pad1(he, E_PAD, 0)
    he_s = _pad1(he, E_PAD, N)

    gidx0 = _prep(jnp.zeros((NC, E_PAD), jnp.int32))
    sidx0 = _prep(jnp.stack([node_s, he_s]))
    degs = _epassdeg(ones_table, gidx0, sidx0, zeros16)
    deg = degs[0, :N, 0]
    bdeg = degs[1, :N, 0]
    Dinv = jnp.where(deg > 0, 1.0 / deg, 0.0)
    Binv = jnp.where(bdeg > 0, 1.0 / bdeg, 0.0)

    g1 = _prep(jnp.stack([2 * node_g, 2 * node_g + 1]))
    s1 = _prep(jnp.stack([he_s, he_s]))
    g2 = _prep(jnp.stack([he_g, he_g + N]))
    s2 = _prep(jnp.stack([node_s, node_s]))

    p1 = _epass64(x.reshape(2 * N, 64), g1, s1, zeros64)
    oe = p1[:, :N] * Binv[None, :, None]
    p2 = _epass64(oe.reshape(2 * N, 64), g2, s2, zeros64)
    pn = p2[:, :N] * Dinv[None, :, None]
    h1 = jax.nn.relu(pn[0] @ W1[:, :64].T + pn[1] @ W1[:, 64:].T + b1)

    xw2 = h1 @ W2.T
    q1 = _epass16(xw2.reshape(2 * N, 16), g1, s1, zeros16)
    oe2 = q1[:, :N] * Binv[None, :, None]
    q2 = _epass16(oe2.reshape(2 * N, 16), g2, s2, zeros16)
    h2h = jax.nn.relu(q2[:, :N] * Dinv[None, :, None] + b2.reshape(2, 1, 16))
    h2 = h2h.transpose(1, 0, 2).reshape(N, 32)

    table = jnp.concatenate([jnp.zeros((1, 32), jnp.float32), h2], axis=0)
    hflat = _pad1(hnode.reshape(-1).astype(jnp.int32), HYE_PAD, 0)
    gh = _prep(jnp.stack([2 * hflat, 2 * hflat + 1]))
    ghye = _epasshye(table.reshape(2 * (N + 1), 16), gh, hye_sidx, zeros16)
    x2 = ghye[:, :N].transpose(1, 0, 2).reshape(N, 32) / (hlen + EPS)
    return h2, x2


def _fusion(z, W1, b1, W2):
    w = (jnp.tanh(z @ W1.T + b1) @ W2.T).mean(0)
    beta = jax.nn.softmax(w, axis=0)
    return (beta[None, :, :] * z).sum(1)


def kernel(x0, x1, x2, x3, edge0, edge1, edge2, edge3, hlen0, hlen1, hlen2, hlen3, hnode0, hnode1, hnode2, hnode3, conv1_W, conv1_b, conv2_W, conv2_b, attnc_W1, attnc_b1, attnc_W2, attnm_W1, attnm_b1, attnm_W2):
    zeros64 = jnp.zeros((ACC_ROWS, 64), jnp.float32)
    zeros16 = jnp.zeros((ACC_ROWS, 16), jnp.float32)
    ones_table = jnp.ones((8, 16), jnp.float32)
    hye_dst = _pad1(jnp.repeat(jnp.arange(N, dtype=jnp.int32), L), HYE_PAD, N)
    hye_sidx = _prep(jnp.stack([hye_dst, hye_dst]))

    args = (conv1_W, conv1_b, conv2_W, conv2_b, zeros64, zeros16, ones_table,
            hye_sidx)
    x1_cm, x2_cm = _graph_pipeline(x0, edge0, hlen0, hnode0, *args)
    x1_mc, x2_mc = _graph_pipeline(x1, edge1, hlen1, hnode1, *args)
    x1_cc, x2_cc = _graph_pipeline(x2, edge2, hlen2, hnode2, *args)
    x1_mm, x2_mm = _graph_pipeline(x3, edge3, hlen3, hnode3, *args)

    x_c = jnp.stack((x1_mc, x2_cm, x1_cc, x2_cc), axis=1)
    x_m = jnp.stack((x1_cm, x2_mc, x1_mm, x2_mm), axis=1)
    h_c = _fusion(x_c, attnc_W1, attnc_b1, attnc_W2)
    h_m = _fusion(x_m, attnm_W1, attnm_b1, attnm_W2)
    return (h_c, h_m)


# speedup vs baseline: 2.1902x; 1.0158x over previous
import functools

import jax
import jax.numpy as jnp
from jax import lax
from jax.experimental import pallas as pl
from jax.experimental.pallas import tpu as pltpu
from jax.experimental.pallas import tpu_sc as plsc

N = 10000
E = 320000
L = 16
EPS = 1e-15

NC = 2
NS = 16
CHUNK = 1024
KSUB = CHUNK // 128
ACC_ROWS = NS * 632
E_PAD = NS * 20 * CHUNK
HYE_PAD = NS * 10 * CHUNK


def _make_epass(table_rows, dh, chunk, nchunks):
    ksub = chunk // 128
    mesh = plsc.VectorSubcoreMesh(
        core_axis_name="c", subcore_axis_name="s", num_cores=NC, num_subcores=NS)

    @functools.partial(
        pl.kernel,
        out_type=jax.ShapeDtypeStruct((NC, ACC_ROWS, dh), jnp.float32),
        mesh=mesh,
        scratch_types=[
            pltpu.VMEM((ksub, 128), jnp.int32),
            pltpu.VMEM((ksub, 128), jnp.int32),
            pltpu.VMEM((ksub, 128), jnp.int32),
            pltpu.VMEM((ksub, 128), jnp.int32),
            pltpu.VMEM((chunk, dh), jnp.float32),
            pltpu.VMEM((chunk, dh), jnp.float32),
            pltpu.VMEM_SHARED((ACC_ROWS, dh), jnp.float32),
            pltpu.SemaphoreType.DMA,
            pltpu.SemaphoreType.DMA,
            pltpu.SemaphoreType.DMA,
            pltpu.SemaphoreType.DMA,
        ],
        compiler_params=pltpu.CompilerParams(use_tc_tiling_on_sc=False),
    )
    def epass(table, gidx, sidx, zeros, out, gia, sia, gib, sib, r_a, r_b,
              acc, sga, sgb, ssa, ssb):
        c = lax.axis_index("c")
        s = lax.axis_index("s")
        zrows = ACC_ROWS // NS
        pltpu.sync_copy(zeros.at[pl.ds(s * zrows, zrows)],
                        acc.at[pl.ds(s * zrows, zrows)])
        plsc.subcore_barrier()

        def chunk_body(i, carry):
            cid = s * nchunks + 2 * i
            pltpu.sync_copy(gidx.at[c, cid], gia)
            pltpu.sync_copy(sidx.at[c, cid], sia)
            ga = [
                pltpu.async_copy(table.at[gia.at[j]],
                                 r_a.at[pl.ds(j * 128, 128)], sga)
                for j in range(ksub)
            ]
            pltpu.sync_copy(gidx.at[c, cid + 1], gib)
            pltpu.sync_copy(sidx.at[c, cid + 1], sib)
            gb = [
                pltpu.async_copy(table.at[gib.at[j]],
                                 r_b.at[pl.ds(j * 128, 128)], sgb)
                for j in range(ksub)
            ]
            for d in ga:
                d.wait()
            sa = [
                pltpu.async_copy(r_a.at[pl.ds(j * 128, 128)],
                                 acc.at[sia.at[j]], ssa, add=True)
                for j in range(ksub)
            ]
            for d in gb:
                d.wait()
            sb = [
                pltpu.async_copy(r_b.at[pl.ds(j * 128, 128)],
                                 acc.at[sib.at[j]], ssb, add=True)
                for j in range(ksub)
            ]
            for d in sa:
                d.wait()
            for d in sb:
                d.wait()
            return carry

        lax.fori_loop(0, nchunks // 2, chunk_body, 0)
        plsc.subcore_barrier()
        pltpu.sync_copy(acc.at[pl.ds(s * zrows, zrows)],
                        out.at[c, pl.ds(s * zrows, zrows)])

    return epass


_epass64 = _make_epass(2 * N, 64, 512, E_PAD // NS // 512)
_epass16 = _make_epass(2 * N, 16, 1024, E_PAD // NS // 1024)
_epassdeg = _make_epass(8, 16, 1024, E_PAD // NS // 1024)
_epasshye = _make_epass(2 * (N + 1), 16, 1024, HYE_PAD // NS // 1024)


def _pad1(a, n, val):
    return jnp.concatenate([a, jnp.full((n - a.shape[0],), val, a.dtype)])


def _prep(idx2, ksub=8):
    return idx2.reshape(NC, -1, ksub, 128)


def _graph_pipeline(x, ei, hlen, hnode, W1, b1, W2, b2, zeros64, zeros16,
                    ones_table, hye_sidx):
    node = ei[0]
    he = ei[1]
    node_g = _pad1(node, E_PAD, 0)
    node_s = _pad1(node, E_PAD, N)
    he_g = _pad1(he, E_PAD, 0)
    he_s = _pad1(he, E_PAD, N)

    gidx0 = _prep(jnp.zeros((NC, E_PAD), jnp.int32))
    sidx0 = _prep(jnp.stack([node_s, he_s]))
    degs = _epassdeg(ones_table, gidx0, sidx0, zeros16)
    deg = degs[0, :N, 0]
    bdeg = degs[1, :N, 0]
    Dinv = jnp.where(deg > 0, 1.0 / deg, 0.0)
    Binv = jnp.where(bdeg > 0, 1.0 / bdeg, 0.0)

    g1 = jnp.stack([2 * node_g, 2 * node_g + 1])
    s1 = jnp.stack([he_s, he_s])
    g2 = jnp.stack([he_g, he_g + N])
    s2 = jnp.stack([node_s, node_s])

    p1 = _epass64(x.reshape(2 * N, 64), _prep(g1, 4), _prep(s1, 4), zeros64)
    oe = p1[:, :N] * Binv[None, :, None]
    p2 = _epass64(oe.reshape(2 * N, 64), _prep(g2, 4), _prep(s2, 4), zeros64)
    pn = p2[:, :N] * Dinv[None, :, None]
    h1 = jax.nn.relu(pn[0] @ W1[:, :64].T + pn[1] @ W1[:, 64:].T + b1)

    xw2 = h1 @ W2.T
    q1 = _epass16(xw2.reshape(2 * N, 16), _prep(g1), _prep(s1), zeros16)
    oe2 = q1[:, :N] * Binv[None, :, None]
    q2 = _epass16(oe2.reshape(2 * N, 16), _prep(g2), _prep(s2), zeros16)
    h2h = jax.nn.relu(q2[:, :N] * Dinv[None, :, None] + b2.reshape(2, 1, 16))
    h2 = h2h.transpose(1, 0, 2).reshape(N, 32)

    table = jnp.concatenate([jnp.zeros((1, 32), jnp.float32), h2], axis=0)
    hflat = _pad1(hnode.reshape(-1).astype(jnp.int32), HYE_PAD, 0)
    gh = _prep(jnp.stack([2 * hflat, 2 * hflat + 1]))
    ghye = _epasshye(table.reshape(2 * (N + 1), 16), gh, hye_sidx, zeros16)
    x2 = ghye[:, :N].transpose(1, 0, 2).reshape(N, 32) / (hlen + EPS)
    return h2, x2


def _fusion(z, W1, b1, W2):
    w = (jnp.tanh(z @ W1.T + b1) @ W2.T).mean(0)
    beta = jax.nn.softmax(w, axis=0)
    return (beta[None, :, :] * z).sum(1)


def kernel(x0, x1, x2, x3, edge0, edge1, edge2, edge3, hlen0, hlen1, hlen2, hlen3, hnode0, hnode1, hnode2, hnode3, conv1_W, conv1_b, conv2_W, conv2_b, attnc_W1, attnc_b1, attnc_W2, attnm_W1, attnm_b1, attnm_W2):
    zeros64 = jnp.zeros((ACC_ROWS, 64), jnp.float32)
    zeros16 = jnp.zeros((ACC_ROWS, 16), jnp.float32)
    ones_table = jnp.ones((8, 16), jnp.float32)
    hye_dst = _pad1(jnp.repeat(jnp.arange(N, dtype=jnp.int32), L), HYE_PAD, N)
    hye_sidx = _prep(jnp.stack([hye_dst, hye_dst]))

    args = (conv1_W, conv1_b, conv2_W, conv2_b, zeros64, zeros16, ones_table,
            hye_sidx)
    x1_cm, x2_cm = _graph_pipeline(x0, edge0, hlen0, hnode0, *args)
    x1_mc, x2_mc = _graph_pipeline(x1, edge1, hlen1, hnode1, *args)
    x1_cc, x2_cc = _graph_pipeline(x2, edge2, hlen2, hnode2, *args)
    x1_mm, x2_mm = _graph_pipeline(x3, edge3, hlen3, hnode3, *args)

    x_c = jnp.stack((x1_mc, x2_cm, x1_cc, x2_cc), axis=1)
    x_m = jnp.stack((x1_cm, x2_mc, x1_mm, x2_mm), axis=1)
    h_c = _fusion(x_c, attnc_W1, attnc_b1, attnc_W2)
    h_m = _fusion(x_m, attnm_W1, attnm_b1, attnm_W2)
    return (h_c, h_m)


# speedup vs baseline: 7.9383x; 3.6244x over previous
import functools

import jax
import jax.numpy as jnp
from jax import lax
from jax.experimental import pallas as pl
from jax.experimental.pallas import tpu as pltpu
from jax.experimental.pallas import tpu_sc as plsc

N = 10000
E = 320000
L = 16
EPS = 1e-15

NC = 2
NS = 16
CHUNK = 1024
KSUB = CHUNK // 128
ACC_ROWS = NS * 632
E_PAD = NS * 20 * CHUNK
HYE_PAD = NS * 10 * CHUNK


def _make_epass(table_rows, dh, chunk, nchunks):
    ksub = chunk // 128
    mesh = plsc.VectorSubcoreMesh(
        core_axis_name="c", subcore_axis_name="s", num_cores=NC, num_subcores=NS)

    @functools.partial(
        pl.kernel,
        out_type=jax.ShapeDtypeStruct((NC, ACC_ROWS, dh), jnp.float32),
        mesh=mesh,
        scratch_types=[
            pltpu.VMEM((ksub, 128), jnp.int32),
            pltpu.VMEM((ksub, 128), jnp.int32),
            pltpu.VMEM((ksub, 128), jnp.int32),
            pltpu.VMEM((ksub, 128), jnp.int32),
            pltpu.VMEM((chunk, dh), jnp.float32),
            pltpu.VMEM((chunk, dh), jnp.float32),
            pltpu.VMEM_SHARED((ACC_ROWS, dh), jnp.float32),
            pltpu.SemaphoreType.DMA,
            pltpu.SemaphoreType.DMA,
            pltpu.SemaphoreType.DMA,
            pltpu.SemaphoreType.DMA,
        ],
        compiler_params=pltpu.CompilerParams(use_tc_tiling_on_sc=False),
    )
    def epass(table, gidx, sidx, zeros, out, gia, sia, gib, sib, r_a, r_b,
              acc, sga, sgb, ssa, ssb):
        c = lax.axis_index("c")
        s = lax.axis_index("s")
        zrows = ACC_ROWS // NS
        pltpu.sync_copy(zeros.at[pl.ds(s * zrows, zrows)],
                        acc.at[pl.ds(s * zrows, zrows)])
        plsc.subcore_barrier()

        def chunk_body(i, carry):
            cid = s * nchunks + 2 * i
            pltpu.sync_copy(gidx.at[c, cid], gia)
            pltpu.sync_copy(sidx.at[c, cid], sia)
            ga = [
                pltpu.async_copy(table.at[gia.at[j]],
                                 r_a.at[pl.ds(j * 128, 128)], sga)
                for j in range(ksub)
            ]
            pltpu.sync_copy(gidx.at[c, cid + 1], gib)
            pltpu.sync_copy(sidx.at[c, cid + 1], sib)
            gb = [
                pltpu.async_copy(table.at[gib.at[j]],
                                 r_b.at[pl.ds(j * 128, 128)], sgb)
                for j in range(ksub)
            ]
            for d in ga:
                d.wait()
            sa = [
                pltpu.async_copy(r_a.at[pl.ds(j * 128, 128)],
                                 acc.at[sia.at[j]], ssa, add=True)
                for j in range(ksub)
            ]
            for d in gb:
                d.wait()
            sb = [
                pltpu.async_copy(r_b.at[pl.ds(j * 128, 128)],
                                 acc.at[sib.at[j]], ssb, add=True)
                for j in range(ksub)
            ]
            for d in sa:
                d.wait()
            for d in sb:
                d.wait()
            return carry

        lax.fori_loop(0, nchunks // 2, chunk_body, 0)
        plsc.subcore_barrier()
        pltpu.sync_copy(acc.at[pl.ds(s * zrows, zrows)],
                        out.at[c, pl.ds(s * zrows, zrows)])

    return epass


def _make_degpass(chunk, nchunks):
    ksub = chunk // 128
    mesh = plsc.VectorSubcoreMesh(
        core_axis_name="c", subcore_axis_name="s", num_cores=NC, num_subcores=NS)

    @functools.partial(
        pl.kernel,
        out_type=jax.ShapeDtypeStruct((NC, ACC_ROWS, 16), jnp.float32),
        mesh=mesh,
        scratch_types=[
            pltpu.VMEM((ksub, 128), jnp.int32),
            pltpu.VMEM((chunk, 16), jnp.float32),
            pltpu.VMEM_SHARED((ACC_ROWS, 16), jnp.float32),
            pltpu.SemaphoreType.DMA,
        ],
        compiler_params=pltpu.CompilerParams(use_tc_tiling_on_sc=False),
    )
    def degpass(ones, sidx, zeros, out, si_v, ones_v, acc, sem):
        c = lax.axis_index("c")
        s = lax.axis_index("s")
        zrows = ACC_ROWS // NS
        pltpu.sync_copy(zeros.at[pl.ds(s * zrows, zrows)],
                        acc.at[pl.ds(s * zrows, zrows)])
        pltpu.sync_copy(ones, ones_v)
        plsc.subcore_barrier()

        def chunk_body(i, carry):
            pltpu.sync_copy(sidx.at[c, s * nchunks + i], si_v)
            descs = [
                pltpu.async_copy(ones_v.at[pl.ds(j * 128, 128)],
                                 acc.at[si_v.at[j]], sem, add=True)
                for j in range(ksub)
            ]
            for d in descs:
                d.wait()
            return carry

        lax.fori_loop(0, nchunks, chunk_body, 0)
        plsc.subcore_barrier()
        pltpu.sync_copy(acc.at[pl.ds(s * zrows, zrows)],
                        out.at[c, pl.ds(s * zrows, zrows)])

    return degpass


_epass64 = _make_epass(2 * N, 64, 512, E_PAD // NS // 512)
_epass16 = _make_epass(2 * N, 16, 1024, E_PAD // NS // 1024)
_epassdeg = _make_degpass(1024, E_PAD // NS // 1024)
_epasshye = _make_epass(2 * (N + 1), 16, 1024, HYE_PAD // NS // 1024)


def _pad1(a, n, val):
    return jnp.concatenate([a, jnp.full((n - a.shape[0],), val, a.dtype)])


def _prep(idx2, ksub=8):
    return idx2.reshape(NC, -1, ksub, 128)


def _graph_pipeline(x, ei, hlen, hnode, W1, b1, W2, b2, zeros64, zeros16,
                    ones_table, hye_sidx):
    node = ei[0]
    he = ei[1]
    node_g = _pad1(node, E_PAD, 0)
    node_s = _pad1(node, E_PAD, N)
    he_g = _pad1(he, E_PAD, 0)
    he_s = _pad1(he, E_PAD, N)

    sidx0 = _prep(jnp.stack([node_s, he_s]))
    degs = _epassdeg(ones_table, sidx0, zeros16)
    deg = degs[0, :N, 0]
    bdeg = degs[1, :N, 0]
    Dinv = jnp.where(deg > 0, 1.0 / deg, 0.0)
    Binv = jnp.where(bdeg > 0, 1.0 / bdeg, 0.0)

    g1 = jnp.stack([2 * node_g, 2 * node_g + 1])
    s1 = jnp.stack([he_s, he_s])
    g2 = jnp.stack([he_g, he_g + N])
    s2 = jnp.stack([node_s, node_s])

    p1 = _epass64(x.reshape(2 * N, 64), _prep(g1, 4), _prep(s1, 4), zeros64)
    oe = p1[:, :N] * Binv[None, :, None]
    p2 = _epass64(oe.reshape(2 * N, 64), _prep(g2, 4), _prep(s2, 4), zeros64)
    pn = p2[:, :N] * Dinv[None, :, None]
    h1 = jax.nn.relu(pn[0] @ W1[:, :64].T + pn[1] @ W1[:, 64:].T + b1)

    xw2 = h1 @ W2.T
    q1 = _epass16(xw2.reshape(2 * N, 16), _prep(g1), _prep(s1), zeros16)
    oe2 = q1[:, :N] * Binv[None, :, None]
    q2 = _epass16(oe2.reshape(2 * N, 16), _prep(g2), _prep(s2), zeros16)
    h2h = jax.nn.relu(q2[:, :N] * Dinv[None, :, None] + b2.reshape(2, 1, 16))
    h2 = h2h.transpose(1, 0, 2).reshape(N, 32)

    table = jnp.concatenate([jnp.zeros((1, 32), jnp.float32), h2], axis=0)
    hflat = _pad1(hnode.reshape(-1).astype(jnp.int32), HYE_PAD, 0)
    gh = _prep(jnp.stack([2 * hflat, 2 * hflat + 1]))
    ghye = _epasshye(table.reshape(2 * (N + 1), 16), gh, hye_sidx, zeros16)
    x2 = ghye[:, :N].transpose(1, 0, 2).reshape(N, 32) / (hlen + EPS)
    return h2, x2


def _fusion(z, W1, b1, W2):
    w = (jnp.tanh(z @ W1.T + b1) @ W2.T).mean(0)
    beta = jax.nn.softmax(w, axis=0)
    return (beta[None, :, :] * z).sum(1)


def kernel(x0, x1, x2, x3, edge0, edge1, edge2, edge3, hlen0, hlen1, hlen2, hlen3, hnode0, hnode1, hnode2, hnode3, conv1_W, conv1_b, conv2_W, conv2_b, attnc_W1, attnc_b1, attnc_W2, attnm_W1, attnm_b1, attnm_W2):
    zeros64 = jnp.zeros((ACC_ROWS, 64), jnp.float32)
    zeros16 = jnp.zeros((ACC_ROWS, 16), jnp.float32)
    ones_table = jnp.ones((1024, 16), jnp.float32)
    hye_dst = _pad1(jnp.repeat(jnp.arange(N, dtype=jnp.int32), L), HYE_PAD, N)
    hye_sidx = _prep(jnp.stack([hye_dst, hye_dst]))

    args = (conv1_W, conv1_b, conv2_W, conv2_b, zeros64, zeros16, ones_table,
            hye_sidx)
    x1_cm, x2_cm = _graph_pipeline(x0, edge0, hlen0, hnode0, *args)
    x1_mc, x2_mc = _graph_pipeline(x1, edge1, hlen1, hnode1, *args)
    x1_cc, x2_cc = _graph_pipeline(x2, edge2, hlen2, hnode2, *args)
    x1_mm, x2_mm = _graph_pipeline(x3, edge3, hlen3, hnode3, *args)

    x_c = jnp.stack((x1_mc, x2_cm, x1_cc, x2_cc), axis=1)
    x_m = jnp.stack((x1_cm, x2_mc, x1_mm, x2_mm), axis=1)
    h_c = _fusion(x_c, attnc_W1, attnc_b1, attnc_W2)
    h_m = _fusion(x_m, attnm_W1, attnm_b1, attnm_W2)
    return (h_c, h_m)
